# Initial kernel scaffold; baseline (speedup 1.0000x reference)
import functools

import jax
import jax.numpy as jnp
from jax import lax
from jax.experimental import pallas as pl
from jax.experimental.pallas import tpu as pltpu
from jax.experimental.pallas import tpu_sc as plsc

N = 10000
E = 320000
R = 8
D = 128
NW = 32
EPW = E // NW
C = 80
NCHUNK = EPW // C
NPW = N // 16
BN = 400


def _embed_body(x_ref, w_ref, b_ref, o_ref):
    o_ref[...] = jax.nn.relu(
        jnp.dot(x_ref[...], w_ref[...], preferred_element_type=jnp.float32)
        + b_ref[...])


def _embed(x, w, b):
    return pl.pallas_call(
        _embed_body,
        grid=(N // BN,),
        in_specs=[
            pl.BlockSpec((BN, D), lambda i: (i, 0)),
            pl.BlockSpec((D, D), lambda i: (0, 0)),
            pl.BlockSpec((1, D), lambda i: (0, 0)),
        ],
        out_specs=pl.BlockSpec((BN, D), lambda i: (i, 0)),
        out_shape=jax.ShapeDtypeStruct((N, D), jnp.float32),
    )(x, w, b.reshape(1, D))


def _dense_body(h_ref, w2_ref, wss_ref, b_ref, wg_ref, am_ref, ad_ref,
                hpg_ref, sn_ref, d_ref, sp_ref):
    h = h_ref[...]
    hp = jnp.dot(h, w2_ref[...], preferred_element_type=jnp.float32)
    g = jax.nn.sigmoid(jnp.sum(h * wg_ref[...], axis=1, keepdims=True))
    hp3 = hp.reshape(BN, R, D)
    sn_ref[...] = jnp.sum(hp3 * am_ref[...].reshape(1, 1, D), axis=2)
    d_ref[...] = jnp.sum(h * ad_ref[...], axis=1, keepdims=True)
    sp_ref[...] = (jnp.dot(h, wss_ref[...], preferred_element_type=jnp.float32)
                   + b_ref[...])
    hpg_ref[...] = hp * g


def _dense(h, w2, wss, b, wg, am, ad):
    return pl.pallas_call(
        _dense_body,
        grid=(N // BN,),
        in_specs=[
            pl.BlockSpec((BN, D), lambda i: (i, 0)),
            pl.BlockSpec((D, R * D), lambda i: (0, 0)),
            pl.BlockSpec((D, D), lambda i: (0, 0)),
            pl.BlockSpec((1, D), lambda i: (0, 0)),
            pl.BlockSpec((1, D), lambda i: (0, 0)),
            pl.BlockSpec((1, D), lambda i: (0, 0)),
            pl.BlockSpec((1, D), lambda i: (0, 0)),
        ],
        out_specs=[
            pl.BlockSpec((BN, R * D), lambda i: (i, 0)),
            pl.BlockSpec((BN, R), lambda i: (i, 0)),
            pl.BlockSpec((BN, 1), lambda i: (i, 0)),
            pl.BlockSpec((BN, D), lambda i: (i, 0)),
        ],
        out_shape=[
            jax.ShapeDtypeStruct((N, R * D), jnp.float32),
            jax.ShapeDtypeStruct((N, R), jnp.float32),
            jax.ShapeDtypeStruct((N, 1), jnp.float32),
            jax.ShapeDtypeStruct((N, D), jnp.float32),
        ],
    )(h, w2, wss, b.reshape(1, D), wg.reshape(1, D), am.reshape(1, D),
      ad.reshape(1, D))


def _edge_body(src_ref, et_ref, fx_ref, hi_ref, lo_ref):
    fx = src_ref[...] * R + et_ref[...]
    fx_ref[...] = fx
    hi_ref[...] = lax.shift_right_logical(fx, 4)
    lo_ref[...] = lax.bitwise_and(fx, 15)


def _edge_pre(src, et):
    e2 = (E // D, D)
    return pl.pallas_call(
        _edge_body,
        grid=(1,),
        in_specs=[pl.BlockSpec(e2, lambda i: (0, 0))] * 2,
        out_specs=[pl.BlockSpec(e2, lambda i: (0, 0))] * 3,
        out_shape=[jax.ShapeDtypeStruct(e2, jnp.int32)] * 3,
    )(src.reshape(e2), et.reshape(e2))


def _epi_body_act(agg_ref, den_ref, sp_ref, o_ref):
    a = agg_ref[0] + agg_ref[1]
    dn = den_ref[0, :, 0:1] + den_ref[1, :, 0:1]
    o_ref[...] = jax.nn.relu(a / (dn + 1e-9) + sp_ref[...])


def _epi_body_noact(agg_ref, den_ref, sp_ref, o_ref):
    a = agg_ref[0] + agg_ref[1]
    dn = den_ref[0, :, 0:1] + den_ref[1, :, 0:1]
    o_ref[...] = a / (dn + 1e-9) + sp_ref[...]


def _epilogue(agg2, den2, sp, act):
    return pl.pallas_call(
        _epi_body_act if act else _epi_body_noact,
        grid=(N // BN,),
        in_specs=[
            pl.BlockSpec((2, BN, D), lambda i: (0, i, 0)),
            pl.BlockSpec((2, BN, 16), lambda i: (0, i, 0)),
            pl.BlockSpec((BN, D), lambda i: (i, 0)),
        ],
        out_specs=pl.BlockSpec((BN, D), lambda i: (i, 0)),
        out_shape=jax.ShapeDtypeStruct((N, D), jnp.float32),
    )(agg2, den2, sp)


_MESH = plsc.VectorSubcoreMesh(core_axis_name="c", subcore_axis_name="s")


@functools.partial(
    pl.kernel,
    out_type=[
        jax.ShapeDtypeStruct((2, N, D), jnp.float32),
        jax.ShapeDtypeStruct((2, N, 16), jnp.float32),
    ],
    mesh=_MESH,
    scratch_types=[
        pltpu.VMEM((N,), jnp.float32),
        pltpu.VMEM((4, C), jnp.int32),
        pltpu.VMEM((C, D), jnp.float32),
        pltpu.VMEM((C, 16), jnp.float32),
        pltpu.VMEM((C, 16), jnp.float32),
        pltpu.VMEM_SHARED((N, D), jnp.float32),
        pltpu.VMEM_SHARED((N, 16), jnp.float32),
    ],
)
def _sc_edge(hproj_hbm, snode_hbm, d_hbm, edata_hbm, agg_out, den_out,
             d_v, ebuf_v, rows_v, sblk_v, denb_v, agg_sh, den_sh):
    core = lax.axis_index("c")
    sub = lax.axis_index("s")
    wid = core * 16 + sub
    iota = lax.iota(jnp.int32, 16)
    zero16 = jnp.zeros((16,), jnp.float32)
    zidx = jnp.zeros((16,), jnp.int32)

    pltpu.sync_copy(d_hbm, d_v)

    @pl.loop(0, C)
    def _(i):
        row = jnp.full((16,), i, jnp.int32)
        plsc.store_scatter(denb_v, [row, iota], zero16)

    @pl.loop(0, 25)
    def _(i):
        row = jnp.full((16,), i, jnp.int32)
        for j in range(8):
            plsc.store_scatter(rows_v, [row, iota + j * 16], zero16)

    nbase = sub * NPW
    @pl.loop(0, 25)
    def _(k):
        pltpu.sync_copy(rows_v.at[pl.ds(0, 25)],
                        agg_sh.at[pl.ds(nbase + k * 25, 25)])
        pltpu.sync_copy(denb_v.at[pl.ds(0, 25)],
                        den_sh.at[pl.ds(nbase + k * 25, 25)])

    plsc.subcore_barrier()

    @pl.loop(0, NCHUNK)
    def _(c):
        pltpu.sync_copy(edata_hbm.at[wid * NCHUNK + c], ebuf_v)
        pltpu.sync_copy(hproj_hbm.at[ebuf_v.at[0]], rows_v)
        pltpu.sync_copy(snode_hbm.at[ebuf_v.at[1]], sblk_v)

        for g in range(C // 16):
            lanes = iota + g * 16
            lo = plsc.load_gather(ebuf_v, [jnp.full((16,), 2, jnp.int32), lanes])
            dstg = plsc.load_gather(ebuf_v, [jnp.full((16,), 3, jnp.int32), lanes])
            s = plsc.load_gather(sblk_v, [lanes, lo])
            dd = plsc.load_gather(d_v, [dstg])
            t = s + dd
            sc = jnp.minimum(jnp.maximum(t, 0.2 * t), 80.0)
            ex = jnp.exp(sc)
            plsc.store_scatter(denb_v, [lanes, zidx], ex)

        @pl.loop(0, C)
        def _(e):
            erow = jnp.full((16,), e, jnp.int32)
            exb = plsc.load_gather(denb_v, [erow, zidx])
            for j in range(8):
                col = iota + j * 16
                v = plsc.load_gather(rows_v, [erow, col])
                plsc.store_scatter(rows_v, [erow, col], v * exb)

        pltpu.sync_copy(rows_v, agg_sh.at[ebuf_v.at[3]], add=True)
        pltpu.sync_copy(denb_v, den_sh.at[ebuf_v.at[3]], add=True)

    plsc.subcore_barrier()

    @pl.loop(0, 5)
    def _(k):
        pltpu.sync_copy(agg_sh.at[pl.ds(nbase + k * 125, 125)],
                        agg_out.at[core, pl.ds(nbase + k * 125, 125)])
        pltpu.sync_copy(den_sh.at[pl.ds(nbase + k * 125, 125)],
                        den_out.at[core, pl.ds(nbase + k * 125, 125)])


def _layer(h, p, edata, act):
    w2 = p['Wr'].transpose(1, 0, 2).reshape(D, R * D)
    wss = p['Wself'] + p['Wskip'] if 'Wskip' in p else p['Wself']
    hpg, sn, d, sp = _dense(h, w2, wss, p['b'], p['wg'], p['am'], p['ad'])
    agg2, den2 = _sc_edge(hpg.reshape(N * R, D), sn.reshape(N * R // 16, 16),
                          d.reshape(N), edata)
    return _epilogue(agg2, den2, sp, act)


def kernel(x, edge_index, edge_type, params):
    src = edge_index[0]
    dst = edge_index[1]
    fx, hi, lo = _edge_pre(src, edge_type)
    edata = jnp.stack(
        [fx.reshape(E), hi.reshape(E), lo.reshape(E), dst],
        axis=0).reshape(4, E // C, C).transpose(1, 0, 2)
    h = _embed(x, params['emb_W'], params['emb_b'])
    h = _layer(h, params['l1'], edata, True)
    h = _layer(h, params['l2'], edata, True)
    h = _layer(h, params['l3'], edata, False)
    return h


# speedup vs baseline: 13.9091x; 13.9091x over previous
import dataclasses
import functools

import jax
import jax.numpy as jnp
from jax import lax
from jax.experimental import pallas as pl
from jax.experimental.pallas import tpu as pltpu
from jax.experimental.pallas import tpu_sc as plsc

N = 10000
E = 320000
R = 8
D = 128
NW = 32
EPW = E // NW
C = 80
NCHUNK = EPW // C
NPAD = 10240
NPW = NPAD // 16
BN = 400


def _embed_body(x_ref, w_ref, b_ref, o_ref):
    o_ref[...] = jax.nn.relu(
        jnp.dot(x_ref[...], w_ref[...], preferred_element_type=jnp.float32)
        + b_ref[...])


def _embed(x, w, b):
    return pl.pallas_call(
        _embed_body,
        grid=(N // BN,),
        in_specs=[
            pl.BlockSpec((BN, D), lambda i: (i, 0)),
            pl.BlockSpec((D, D), lambda i: (0, 0)),
            pl.BlockSpec((1, D), lambda i: (0, 0)),
        ],
        out_specs=pl.BlockSpec((BN, D), lambda i: (i, 0)),
        out_shape=jax.ShapeDtypeStruct((N, D), jnp.float32),
    )(x, w, b.reshape(1, D))


def _dense_body(h_ref, w2_ref, wss_ref, b_ref, wg_ref, am_ref, ad_ref,
                hpg_ref, sn_ref, d_ref, sp_ref):
    h = h_ref[...]
    hp = jnp.dot(h, w2_ref[...], preferred_element_type=jnp.float32)
    g = jax.nn.sigmoid(jnp.sum(h * wg_ref[...], axis=1, keepdims=True))
    hp3 = hp.reshape(BN, R, D)
    sn_ref[...] = jnp.sum(hp3 * am_ref[...].reshape(1, 1, D), axis=2)
    d_ref[...] = jnp.sum(h * ad_ref[...], axis=1, keepdims=True)
    sp_ref[...] = (jnp.dot(h, wss_ref[...], preferred_element_type=jnp.float32)
                   + b_ref[...])
    hpg_ref[...] = hp * g


def _dense(h, w2, wss, b, wg, am, ad):
    return pl.pallas_call(
        _dense_body,
        grid=(N // BN,),
        in_specs=[
            pl.BlockSpec((BN, D), lambda i: (i, 0)),
            pl.BlockSpec((D, R * D), lambda i: (0, 0)),
            pl.BlockSpec((D, D), lambda i: (0, 0)),
            pl.BlockSpec((1, D), lambda i: (0, 0)),
            pl.BlockSpec((1, D), lambda i: (0, 0)),
            pl.BlockSpec((1, D), lambda i: (0, 0)),
            pl.BlockSpec((1, D), lambda i: (0, 0)),
        ],
        out_specs=[
            pl.BlockSpec((BN, R * D), lambda i: (i, 0)),
            pl.BlockSpec((BN, R), lambda i: (i, 0)),
            pl.BlockSpec((BN, 1), lambda i: (i, 0)),
            pl.BlockSpec((BN, D), lambda i: (i, 0)),
        ],
        out_shape=[
            jax.ShapeDtypeStruct((N, R * D), jnp.float32),
            jax.ShapeDtypeStruct((N, R), jnp.float32),
            jax.ShapeDtypeStruct((N, 1), jnp.float32),
            jax.ShapeDtypeStruct((N, D), jnp.float32),
        ],
    )(h, w2, wss, b.reshape(1, D), wg.reshape(1, D), am.reshape(1, D),
      ad.reshape(1, D))


def _edge_body(src_ref, et_ref, fx_ref, hi_ref, lo_ref):
    fx = src_ref[...] * R + et_ref[...]
    fx_ref[...] = fx
    hi_ref[...] = lax.shift_right_logical(fx, 4)
    lo_ref[...] = lax.bitwise_and(fx, 15)


def _edge_pre(src, et):
    e2 = (E // D, D)
    return pl.pallas_call(
        _edge_body,
        grid=(1,),
        in_specs=[pl.BlockSpec(e2, lambda i: (0, 0))] * 2,
        out_specs=[pl.BlockSpec(e2, lambda i: (0, 0))] * 3,
        out_shape=[jax.ShapeDtypeStruct(e2, jnp.int32)] * 3,
    )(src.reshape(e2), et.reshape(e2))


def _epi_body_act(agg_ref, den_ref, sp_ref, o_ref):
    a = agg_ref[0] + agg_ref[1]
    dn = den_ref[0, :, 0:1] + den_ref[1, :, 0:1]
    o_ref[...] = jax.nn.relu(a / (dn + 1e-9) + sp_ref[...])


def _epi_body_noact(agg_ref, den_ref, sp_ref, o_ref):
    a = agg_ref[0] + agg_ref[1]
    dn = den_ref[0, :, 0:1] + den_ref[1, :, 0:1]
    o_ref[...] = a / (dn + 1e-9) + sp_ref[...]


def _epilogue(agg2, den2, sp, act):
    return pl.pallas_call(
        _epi_body_act if act else _epi_body_noact,
        grid=(N // BN,),
        in_specs=[
            pl.BlockSpec((2, BN, D), lambda i: (0, i, 0)),
            pl.BlockSpec((2, BN, 16), lambda i: (0, i, 0)),
            pl.BlockSpec((BN, D), lambda i: (i, 0)),
        ],
        out_specs=pl.BlockSpec((BN, D), lambda i: (i, 0)),
        out_shape=jax.ShapeDtypeStruct((N, D), jnp.float32),
    )(agg2, den2, sp)


_MESH = plsc.VectorSubcoreMesh(core_axis_name="c", subcore_axis_name="s")

_CP = pltpu.CompilerParams()
if "needs_layout_passes" in pltpu.CompilerParams.__dataclass_fields__:
    _CP = dataclasses.replace(_CP, needs_layout_passes=False)
if "use_tc_tiling_on_sc" in pltpu.CompilerParams.__dataclass_fields__:
    _CP = dataclasses.replace(_CP, use_tc_tiling_on_sc=False)


@functools.partial(
    pl.kernel,
    compiler_params=_CP,
    out_type=[
        jax.ShapeDtypeStruct((2, NPAD, D), jnp.float32),
        jax.ShapeDtypeStruct((2, NPAD, 16), jnp.float32),
    ],
    mesh=_MESH,
    scratch_types=[
        pltpu.VMEM((N,), jnp.float32),
        pltpu.VMEM((4, C), jnp.int32),
        pltpu.VMEM((C, D), jnp.float32),
        pltpu.VMEM((C, 16), jnp.float32),
        pltpu.VMEM((C, 16), jnp.float32),
        pltpu.VMEM_SHARED((NPAD, D), jnp.float32),
        pltpu.VMEM_SHARED((NPAD, 16), jnp.float32),
    ],
)
def _sc_edge(hproj_hbm, snode_hbm, d_hbm, edata_hbm, agg_out, den_out,
             d_v, ebuf_v, rows_v, sblk_v, denb_v, agg_sh, den_sh):
    core = lax.axis_index("c")
    sub = lax.axis_index("s")
    wid = core * 16 + sub
    iota = lax.iota(jnp.int32, 16)
    zero16 = jnp.zeros((16,), jnp.float32)
    zidx = jnp.zeros((16,), jnp.int32)

    pltpu.sync_copy(d_hbm, d_v)

    @pl.loop(0, C)
    def _(i):
        row = jnp.full((16,), i, jnp.int32)
        plsc.store_scatter(denb_v, [row, iota], zero16)

    @pl.loop(0, C)
    def _(i):
        row = jnp.full((16,), i, jnp.int32)
        for j in range(8):
            plsc.store_scatter(rows_v, [row, iota + j * 16], zero16)

    nbase = sub * NPW
    @pl.loop(0, NPW // C)
    def _(k):
        pltpu.sync_copy(rows_v, agg_sh.at[pl.ds(nbase + k * C, C)])
        pltpu.sync_copy(denb_v, den_sh.at[pl.ds(nbase + k * C, C)])

    plsc.subcore_barrier()

    @pl.loop(0, NCHUNK)
    def _(c):
        pltpu.sync_copy(edata_hbm.at[wid * NCHUNK + c], ebuf_v)
        pltpu.sync_copy(hproj_hbm.at[ebuf_v.at[0]], rows_v)
        pltpu.sync_copy(snode_hbm.at[ebuf_v.at[1]], sblk_v)

        for g in range(C // 16):
            lanes = iota + g * 16
            lo = plsc.load_gather(ebuf_v, [jnp.full((16,), 2, jnp.int32), lanes])
            dstg = plsc.load_gather(ebuf_v, [jnp.full((16,), 3, jnp.int32), lanes])
            s = plsc.load_gather(sblk_v, [lanes, lo])
            dd = plsc.load_gather(d_v, [dstg])
            t = s + dd
            sc = jnp.minimum(jnp.maximum(t, 0.2 * t), 80.0)
            ex = jnp.exp(sc)
            plsc.store_scatter(denb_v, [lanes, zidx], ex)

        @pl.loop(0, C)
        def _(e):
            erow = jnp.full((16,), e, jnp.int32)
            exb = plsc.load_gather(denb_v, [erow, zidx])
            for j in range(8):
                col = iota + j * 16
                v = plsc.load_gather(rows_v, [erow, col])
                plsc.store_scatter(rows_v, [erow, col], v * exb)

        pltpu.sync_copy(rows_v, agg_sh.at[ebuf_v.at[3]], add=True)
        pltpu.sync_copy(denb_v, den_sh.at[ebuf_v.at[3]], add=True)

    plsc.subcore_barrier()

    @pl.loop(0, NPW // C)
    def _(k):
        pltpu.sync_copy(agg_sh.at[pl.ds(nbase + k * C, C)],
                        agg_out.at[core, pl.ds(nbase + k * C, C)])
        pltpu.sync_copy(den_sh.at[pl.ds(nbase + k * C, C)],
                        den_out.at[core, pl.ds(nbase + k * C, C)])


def _layer(h, p, edata, act):
    w2 = p['Wr'].transpose(1, 0, 2).reshape(D, R * D)
    wss = p['Wself'] + p['Wskip'] if 'Wskip' in p else p['Wself']
    hpg, sn, d, sp = _dense(h, w2, wss, p['b'], p['wg'], p['am'], p['ad'])
    agg2, den2 = _sc_edge(hpg.reshape(N * R, D), sn.reshape(N * R // 16, 16),
                          d.reshape(N), edata)
    return _epilogue(agg2, den2, sp, act)


def kernel(x, edge_index, edge_type, params):
    src = edge_index[0]
    dst = edge_index[1]
    fx, hi, lo = _edge_pre(src, edge_type)
    edata = jnp.stack(
        [fx.reshape(E), hi.reshape(E), lo.reshape(E), dst],
        axis=0).reshape(4, E // C, C).transpose(1, 0, 2)
    h = _embed(x, params['emb_W'], params['emb_b'])
    h = _layer(h, params['l1'], edata, True)
    h = _layer(h, params['l2'], edata, True)
    h = _layer(h, params['l3'], edata, False)
    return h
